# Initial kernel scaffold; baseline (speedup 1.0000x reference)
import jax
import jax.numpy as jnp
from jax.experimental import pallas as pl


def kernel(batch_user, batch_video, u_embed, v_embed, u_feat, v_feat, word_emb, words_seg, words_id, W_t, b_t, v_base):
    raise NotImplementedError("write your pallas kernel here")


# speedup vs baseline: 11.4921x; 11.4921x over previous
import jax
import jax.numpy as jnp
from jax import lax
from jax.experimental import pallas as pl
from jax.experimental.pallas import tpu as pltpu
from jax.experimental.pallas import tpu_sc as plsc

NUM_USER = 100000
NUM_VIDEO = 50000
DIM_V = 128
LATENT = 64
N_WORDS = 500000
WORD_DIM = 128
BATCH = 4096

NC = 2
NS = 16
L = 16
NTILE = NC * NS
BLK = 1024
NBLK = (N_WORDS + BLK - 1) // BLK
LAST_BLK = N_WORDS - (NBLK - 1) * BLK
PASS_BLKS = 8
MAXM = PASS_BLKS * BLK + 64
RB = 32
INV_SIZE = NUM_VIDEO
DUMMY = BATCH
ACC_ROWS = 4112
SPT = BATCH // NS
EPT = BATCH // NTILE
SHIFT = 8192


def _sc_body(bu_hbm, bvid_hbm, seg_hbm, wid_hbm, wemb_hbm,
             ue_hbm, uf_hbm, ve_hbm, vf_hbm, vb_hbm,
             t_o, cnt_o, bue_o, buf_o, bve_o, vbg_o, bvf_o,
             inv, bv_buf, bu_c, seg_blk, id_blk, packed, cnt_buf,
             rows_a, rows_b, g64, vb1, idxb_a, idxb_b, idxg_a, idxg_b,
             rep_buf, cout, cstage, acc_sh, cnt_sh, csum_sh, sem):
  c = lax.axis_index("c")
  s = lax.axis_index("s")
  wid = c * NS + s
  lane = lax.iota(jnp.int32, L)
  zeros_f = jnp.zeros((L,), jnp.float32)
  zeros_i = jnp.zeros((L,), jnp.int32)
  ones_i = jnp.ones((L,), jnp.int32)

  def zr(r, _):
    for j in range(WORD_DIM // L):
      rows_a[r, pl.ds(j * L, L)] = zeros_f
    return 0
  lax.fori_loop(0, RB, zr, 0)
  base_a = s * (ACC_ROWS // NS)
  for q in range(256 // RB):
    pltpu.sync_copy(rows_a, acc_sh.at[pl.ds(base_a + q * RB, RB)])
  pltpu.sync_copy(rows_a.at[pl.ds(0, 1)], acc_sh.at[pl.ds(base_a + 256, 1)])

  def zc(i, _):
    cnt_buf[pl.ds(i * L, L)] = zeros_i
    return 0
  lax.fori_loop(0, BATCH // L, zc, 0)
  for j in range(RB // L):
    idxg_a[pl.ds(j * L, L)] = zeros_i
    idxg_b[pl.ds(j * L, L)] = zeros_i

  neg1 = jnp.full((L,), -1, jnp.int32)
  def zi(i, _):
    inv[pl.ds(i * L, L)] = neg1
    return 0
  lax.fori_loop(0, INV_SIZE // L, zi, 0)
  pltpu.sync_copy(bvid_hbm, bv_buf)
  def bi(i, _):
    v16 = bv_buf[pl.ds(i * L, L)] - NUM_USER
    bv_buf[pl.ds(i * L, L)] = v16
    plsc.store_scatter(inv, [v16], i * L + lane)
    return 0
  lax.fori_loop(0, BATCH // L, bi, 0)

  plsc.subcore_barrier()

  nblk = (NBLK - 1 - wid) // NTILE + 1
  dum = jnp.full((L,), DUMMY, jnp.int32)

  def scan_flush(i_lo, i_hi):
    def blk_body(i, off):
      bid = wid + i * NTILE
      pltpu.sync_copy(seg_hbm.at[pl.ds(bid * BLK, BLK)], seg_blk)
      pltpu.sync_copy(wid_hbm.at[pl.ds(bid * BLK, BLK)], id_blk)
      bsteps = jnp.where(bid == NBLK - 1, LAST_BLK // L, BLK // L)
      def cstep(k, off2):
        s16 = seg_blk[pl.ds(k * L, L)]
        i16 = id_blk[pl.ds(k * L, L)]
        slot = plsc.load_gather(inv, [s16])
        m = slot >= 0
        slotc = jnp.maximum(slot, 0)
        plsc.addupdate_scatter(cnt_buf, [slotc], ones_i, mask=m)
        plsc.store_compressed(packed.at[pl.ds(off2, L)],
                              i16 * SHIFT + slotc, mask=m)
        n = plsc.all_reduce_population_count(m)
        return off2 + jnp.max(n)
      return lax.fori_loop(0, bsteps, cstep, off)
    m_cnt = lax.fori_loop(i_lo, i_hi, blk_body, jnp.int32(0))

    for j in range(RB // L):
      packed[pl.ds(m_cnt + j * L, L)] = dum

    nwin = (m_cnt + (RB - 1)) // RB
    bufs = ((rows_a, idxb_a, idxg_a), (rows_b, idxb_b, idxg_b))
    def wpair(p, _):
      w = p * 2
      descs = []
      for b in range(2):
        rws, ib, ig = bufs[b]
        @pl.when(w + b < nwin)
        def _():
          base = (w + b) * RB
          for j in range(RB // L):
            p16 = packed[pl.ds(base + j * L, L)]
            ib[pl.ds(j * L, L)] = jnp.bitwise_and(p16, SHIFT - 1)
            ig[pl.ds(j * L, L)] = p16 // SHIFT
        descs.append(pltpu.async_copy(wemb_hbm.at[ig], rws, sem))
      for b in range(2):
        rws, ib, ig = bufs[b]
        descs[b].wait()
        @pl.when(w + b < nwin)
        def _():
          pltpu.sync_copy(rws, acc_sh.at[ib], add=True)
      return 0
    lax.fori_loop(0, (nwin + 1) // 2, wpair, 0)

  scan_flush(0, jnp.minimum(nblk, PASS_BLKS))
  scan_flush(jnp.int32(PASS_BLKS), nblk)

  plsc.subcore_barrier()

  pltpu.sync_copy(cnt_buf, cnt_sh.at[s])
  plsc.subcore_barrier()
  for qt in range(4):
    for t in range(NS):
      pltpu.sync_copy(cnt_sh.at[t, pl.ds(s * SPT + qt * 64, 64)],
                      cstage.at[t])
    for j in range(64 // L):
      acc16 = zeros_i
      for t in range(NS):
        acc16 = acc16 + cstage[t, pl.ds(j * L, L)]
      cout[pl.ds(qt * 64 + j * L, L)] = acc16
  pltpu.sync_copy(cout, csum_sh.at[pl.ds(s * SPT, SPT)])
  plsc.subcore_barrier()
  pltpu.sync_copy(csum_sh, cnt_buf)

  def rstep(k, _):
    v16 = bv_buf[pl.ds(s * SPT + k * L, L)]
    rep16 = plsc.load_gather(inv, [v16])
    rep_buf[pl.ds(k * L, L)] = rep16
    return 0
  lax.fori_loop(0, SPT // L, rstep, 0)
  for h in range(SPT // RB):
    for j in range(RB // L):
      idxb_a[pl.ds(j * L, L)] = rep_buf[pl.ds(h * RB + j * L, L)]
    pltpu.sync_copy(acc_sh.at[idxb_a], rows_a)
    pltpu.sync_copy(rows_a,
                    t_o.at[pl.ds(c * BATCH + s * SPT + h * RB, RB)])
  def cstep2(k, _):
    rep16 = rep_buf[pl.ds(k * L, L)]
    c16 = plsc.load_gather(cnt_buf, [rep16])
    cout[pl.ds(k * L, L)] = c16
    return 0
  lax.fori_loop(0, SPT // L, cstep2, 0)
  pltpu.sync_copy(cout, cnt_o.at[pl.ds(c * BATCH + s * SPT, SPT)])

  ebase = wid * EPT
  pltpu.sync_copy(bu_hbm.at[pl.ds(ebase, EPT)], bu_c)
  def row_gather(table_hbm, idx_buf, idx_off, dst_buf, out_hbm):
    for half in range(EPT // RB):
      def g6(g, _):
        b = g * L
        i16 = idx_buf[pl.ds(idx_off + half * RB + b, L)]
        ds_list = []
        for l in range(L):
          r = i16[l]
          ds_list.append(pltpu.async_copy(
              table_hbm.at[pl.ds(r, 1)], dst_buf.at[pl.ds(b + l, 1)], sem))
        for d in ds_list:
          d.wait()
        return 0
      lax.fori_loop(0, RB // L, g6, 0)
      pltpu.sync_copy(dst_buf, out_hbm.at[pl.ds(ebase + half * RB, RB)])
  row_gather(ue_hbm, bu_c, 0, g64, bue_o)
  row_gather(uf_hbm, bu_c, 0, g64, buf_o)
  row_gather(ve_hbm, bv_buf, ebase, g64, bve_o)
  row_gather(vb_hbm, bv_buf, ebase, vb1, vbg_o)

  for half in range(EPT // RB):
    for j in range(RB // L):
      idxg_a[pl.ds(j * L, L)] = bv_buf[pl.ds(ebase + half * RB + j * L, L)]
    pltpu.async_copy(vf_hbm.at[idxg_a], rows_a, sem).wait()
    pltpu.sync_copy(rows_a, bvf_o.at[pl.ds(ebase + half * RB, RB)])


_SC_OUTS = (
    jax.ShapeDtypeStruct((NC * BATCH, WORD_DIM), jnp.float32),
    jax.ShapeDtypeStruct((NC * BATCH,), jnp.int32),
    jax.ShapeDtypeStruct((BATCH, LATENT), jnp.float32),
    jax.ShapeDtypeStruct((BATCH, LATENT), jnp.float32),
    jax.ShapeDtypeStruct((BATCH, LATENT), jnp.float32),
    jax.ShapeDtypeStruct((BATCH, 1), jnp.float32),
    jax.ShapeDtypeStruct((BATCH, DIM_V), jnp.float32),
)

_SC_SCRATCH = [
    pltpu.VMEM((INV_SIZE,), jnp.int32),
    pltpu.VMEM((BATCH,), jnp.int32),
    pltpu.VMEM((EPT,), jnp.int32),
    pltpu.VMEM((BLK,), jnp.int32),
    pltpu.VMEM((BLK,), jnp.int32),
    pltpu.VMEM((MAXM,), jnp.int32),
    pltpu.VMEM((BATCH,), jnp.int32),
    pltpu.VMEM((RB, WORD_DIM), jnp.float32),
    pltpu.VMEM((RB, WORD_DIM), jnp.float32),
    pltpu.VMEM((RB, LATENT), jnp.float32),
    pltpu.VMEM((RB, 1), jnp.float32),
    pltpu.VMEM((RB,), jnp.int32),
    pltpu.VMEM((RB,), jnp.int32),
    pltpu.VMEM((RB,), jnp.int32),
    pltpu.VMEM((RB,), jnp.int32),
    pltpu.VMEM((SPT,), jnp.int32),
    pltpu.VMEM((SPT,), jnp.int32),
    pltpu.VMEM((NS, 64), jnp.int32),
    pltpu.VMEM_SHARED((ACC_ROWS, WORD_DIM), jnp.float32),
    pltpu.VMEM_SHARED((NS, BATCH), jnp.int32),
    pltpu.VMEM_SHARED((BATCH,), jnp.int32),
    pltpu.SemaphoreType.DMA,
]

_sc_call = pl.kernel(
    _sc_body,
    out_type=_SC_OUTS,
    mesh=plsc.VectorSubcoreMesh(core_axis_name="c", subcore_axis_name="s"),
    scratch_types=_SC_SCRATCH,
    compiler_params=pltpu.CompilerParams(needs_layout_passes=False),
)


def _tc_body(t_r, c2_r, bue_r, buf_r, bve_r, vb_r, bvf_r, w_r, b_r, o_r):
  cnt = jnp.maximum(c2_r[:, 0:1] + c2_r[:, 1:2], 1.0)
  t = (t_r[0:BATCH, :] + t_r[BATCH:2 * BATCH, :]) / cnt
  w1 = w_r[0:DIM_V, :]
  w2 = w_r[DIM_V:2 * DIM_V, :]
  h = (jnp.dot(bvf_r[...], w1, preferred_element_type=jnp.float32)
       + jnp.dot(t, w2, preferred_element_type=jnp.float32)
       + b_r[...])
  h = jnp.maximum(h, 0.0)
  se = jnp.sum(bue_r[...] * bve_r[...], axis=1, keepdims=True)
  sf = jnp.sum(buf_r[...] * h, axis=1, keepdims=True)
  o_r[...] = se + sf + vb_r[...]


_tc_call = pl.pallas_call(
    _tc_body,
    out_shape=jax.ShapeDtypeStruct((BATCH, 1), jnp.float32),
)


def kernel(batch_user, batch_video, u_embed, v_embed, u_feat, v_feat,
           word_emb, words_seg, words_id, W_t, b_t, v_base):
  t_flat, cnt_flat, bue, bufe, bve, vbg, bvf = _sc_call(
      batch_user, batch_video, words_seg, words_id,
      word_emb, u_embed, u_feat, v_embed, v_feat, v_base)
  cnt2 = cnt_flat.reshape(NC, BATCH).astype(jnp.float32).T
  return _tc_call(t_flat, cnt2, bue, bufe, bve, vbg, bvf, W_t,
                  b_t.reshape(1, LATENT))
